# Initial kernel scaffold; baseline (speedup 1.0000x reference)
import jax
import jax.numpy as jnp
from jax.experimental import pallas as pl


def kernel(x, edge_index, W1a, b1a, W2a, b2a, W1m, b1m, W2m, b2m, W1s, b1s, W2s, b2s):
    raise NotImplementedError("write your pallas kernel here")


# speedup vs baseline: 8.8030x; 8.8030x over previous
import functools

import jax
import jax.numpy as jnp
from jax import lax
from jax.experimental import pallas as pl
from jax.experimental.pallas import tpu as pltpu
from jax.experimental.pallas import tpu_sc as plsc

_N = 10000
_E = 320000

_NC = 2
_NS = 16
_NW = _NC * _NS
_B = 80
_EW = _E // _NW
_K = _EW // _B
_RPS = _N // _NS


def _edge_agg(table, srcs, dsts, zeros, d):
  mesh = plsc.VectorSubcoreMesh(core_axis_name="c", subcore_axis_name="s")

  @functools.partial(
      pl.kernel,
      out_type=jax.ShapeDtypeStruct((_NC, _NS, _RPS, d), jnp.float32),
      mesh=mesh,
      compiler_params=pltpu.CompilerParams(use_tc_tiling_on_sc=False),
      scratch_types=[
          pltpu.VMEM((_K, _B), jnp.int32),
          pltpu.VMEM((_K, _B), jnp.int32),
          pltpu.VMEM((2, _B, d), jnp.float32),
          pltpu.VMEM_SHARED((_N, d), jnp.float32),
      ],
  )
  def agg(table_hbm, srcs_hbm, dsts_hbm, zeros_hbm, out_hbm,
          src_v, dst_v, rows_v, acc_sh):
    c = lax.axis_index("c")
    s = lax.axis_index("s")
    wid = s * _NC + c

    pltpu.sync_copy(zeros_hbm, acc_sh.at[pl.ds(s * _RPS, _RPS)])
    pltpu.sync_copy(srcs_hbm.at[wid], src_v)
    pltpu.sync_copy(dsts_hbm.at[wid], dst_v)
    plsc.subcore_barrier()

    @pl.loop(0, _K)
    def _(j):
      pltpu.sync_copy(table_hbm.at[src_v.at[j]], rows_v.at[0])
      pltpu.sync_copy(rows_v.at[0], acc_sh.at[dst_v.at[j]], add=True)

    plsc.subcore_barrier()
    pltpu.sync_copy(acc_sh.at[pl.ds(s * _RPS, _RPS)], out_hbm.at[c, s])

  return agg(table, srcs, dsts, zeros).reshape(_NC, _N, d)


def _dot(a, b):
  return jnp.dot(a, b, preferred_element_type=jnp.float32)


def _proj(x, w):
  def body(x_ref, w_ref, o_ref):
    o_ref[...] = _dot(x_ref[...], w_ref[...])
  return pl.pallas_call(
      body,
      out_shape=jax.ShapeDtypeStruct((x.shape[0], w.shape[1]), jnp.float32),
  )(x, w)


def _mid(z, p0, p1, b1, w2, b2):
  def body(z_ref, p0_ref, p1_ref, b1_ref, w2_ref, b2_ref, o_ref):
    t = jnp.maximum(z_ref[...] + p0_ref[...] + p1_ref[...] + b1_ref[...], 0.0)
    o_ref[...] = jnp.maximum(_dot(t, w2_ref[...]) + b2_ref[...], 0.0)
  return pl.pallas_call(
      body,
      out_shape=jax.ShapeDtypeStruct((z.shape[0], w2.shape[1]), jnp.float32),
  )(z, p0, p1, b1, w2, b2)


def _heads(h, q0, q1, w1m, b1m, w2m, b2m, w1s, b1s, w2s, b2s):
  def body(h_ref, q0_ref, q1_ref, w1m_ref, b1m_ref, w2m_ref, b2m_ref,
           w1s_ref, b1s_ref, w2s_ref, b2s_ref, mu_ref, ls_ref):
    u = h_ref[...] + q0_ref[...] + q1_ref[...]
    tm = jnp.maximum(_dot(u, w1m_ref[...]) + b1m_ref[...], 0.0)
    mu_ref[...] = _dot(tm, w2m_ref[...]) + b2m_ref[...]
    ts = jnp.maximum(_dot(u, w1s_ref[...]) + b1s_ref[...], 0.0)
    ls_ref[...] = _dot(ts, w2s_ref[...]) + b2s_ref[...]
  n = h.shape[0]
  return pl.pallas_call(
      body,
      out_shape=(
          jax.ShapeDtypeStruct((n, w2m.shape[1]), jnp.float32),
          jax.ShapeDtypeStruct((n, w2s.shape[1]), jnp.float32),
      ),
  )(h, q0, q1, w1m, b1m, w2m, b2m, w1s, b1s, w2s, b2s)


@jax.jit
def kernel(x, edge_index, W1a, b1a, W2a, b2a, W1m, b1m, W2m, b2m,
           W1s, b1s, W2s, b2s):
  d_hid = W1a.shape[1]
  d_out = W2a.shape[1]
  srcs = edge_index[0].reshape(_NW, _K, _B)
  dsts = edge_index[1].reshape(_NW, _K, _B)

  z = _proj(x, W1a)
  p = _edge_agg(z, srcs, dsts, jnp.zeros((_RPS, d_hid), jnp.float32), d_hid)
  h = _mid(z, p[0], p[1], b1a.reshape(1, -1), W2a, b2a.reshape(1, -1))

  q = _edge_agg(h, srcs, dsts, jnp.zeros((_RPS, d_out), jnp.float32), d_out)
  mu, logstd = _heads(h, q[0], q[1],
                      W1m, b1m.reshape(1, -1), W2m, b2m.reshape(1, -1),
                      W1s, b1s.reshape(1, -1), W2s, b2s.reshape(1, -1))
  return (mu, logstd)


# speedup vs baseline: 15.5606x; 1.7677x over previous
import functools

import jax
import jax.numpy as jnp
from jax import lax
from jax.experimental import pallas as pl
from jax.experimental.pallas import tpu as pltpu
from jax.experimental.pallas import tpu_sc as plsc

_N = 10000
_E = 320000

_NC = 2
_NS = 16
_NW = _NC * _NS
_B = 125
_EW = _E // _NW
_K = _EW // _B
_RPS = _N // _NS


def _edge_agg(table, srcs, dsts, zeros, d):
  mesh = plsc.VectorSubcoreMesh(core_axis_name="c", subcore_axis_name="s")
  table = table.reshape(_NS, _RPS, d)

  @functools.partial(
      pl.kernel,
      out_type=jax.ShapeDtypeStruct((_NC, _NS, _RPS, d), jnp.float32),
      mesh=mesh,
      compiler_params=pltpu.CompilerParams(use_tc_tiling_on_sc=False),
      scratch_types=[
          pltpu.VMEM((_K, _B), jnp.int32),
          pltpu.VMEM((_K, _B), jnp.int32),
          pltpu.VMEM((2, _B, d), jnp.float32),
          pltpu.VMEM_SHARED((_N, d), jnp.float32),
          pltpu.VMEM_SHARED((_N, d), jnp.float32),
          pltpu.SemaphoreType.DMA,
      ],
  )
  def agg(table_hbm, srcs_hbm, dsts_hbm, zeros_hbm, out_hbm,
          src_v, dst_v, rows_v, tbl_sh, acc_sh, gsem):
    c = lax.axis_index("c")
    s = lax.axis_index("s")
    wid = s * _NC + c

    pltpu.sync_copy(zeros_hbm, acc_sh.at[pl.ds(s * _RPS, _RPS)])
    pltpu.sync_copy(table_hbm.at[s], tbl_sh.at[pl.ds(s * _RPS, _RPS)])
    pltpu.sync_copy(srcs_hbm.at[wid], src_v)
    pltpu.sync_copy(dsts_hbm.at[wid], dst_v)
    plsc.subcore_barrier()

    def wait_gather(b):
      pltpu.make_async_copy(zeros_hbm.at[pl.ds(0, _B)], rows_v.at[b],
                            gsem).wait()

    pltpu.async_copy(tbl_sh.at[src_v.at[0]], rows_v.at[0], gsem)

    @pl.loop(0, _K, step=2)
    def _(j):
      pltpu.async_copy(tbl_sh.at[src_v.at[j + 1]], rows_v.at[1], gsem)
      wait_gather(0)
      pltpu.sync_copy(rows_v.at[0], acc_sh.at[dst_v.at[j]], add=True)

      @pl.when(j + 2 < _K)
      def _():
        pltpu.async_copy(tbl_sh.at[src_v.at[j + 2]], rows_v.at[0], gsem)
      wait_gather(1)
      pltpu.sync_copy(rows_v.at[1], acc_sh.at[dst_v.at[j + 1]], add=True)

    plsc.subcore_barrier()
    pltpu.sync_copy(acc_sh.at[pl.ds(s * _RPS, _RPS)], out_hbm.at[c, s])

  return agg(table, srcs, dsts, zeros).reshape(_NC, _N, d)


def _dot(a, b):
  return jnp.dot(a, b, preferred_element_type=jnp.float32)


def _proj(x, w):
  def body(x_ref, w_ref, o_ref):
    o_ref[...] = _dot(x_ref[...], w_ref[...])
  return pl.pallas_call(
      body,
      out_shape=jax.ShapeDtypeStruct((x.shape[0], w.shape[1]), jnp.float32),
  )(x, w)


def _mid(z, p0, p1, b1, w2, b2):
  def body(z_ref, p0_ref, p1_ref, b1_ref, w2_ref, b2_ref, o_ref):
    t = jnp.maximum(z_ref[...] + p0_ref[...] + p1_ref[...] + b1_ref[...], 0.0)
    o_ref[...] = jnp.maximum(_dot(t, w2_ref[...]) + b2_ref[...], 0.0)
  return pl.pallas_call(
      body,
      out_shape=jax.ShapeDtypeStruct((z.shape[0], w2.shape[1]), jnp.float32),
  )(z, p0, p1, b1, w2, b2)


def _heads(h, q0, q1, w1m, b1m, w2m, b2m, w1s, b1s, w2s, b2s):
  def body(h_ref, q0_ref, q1_ref, w1m_ref, b1m_ref, w2m_ref, b2m_ref,
           w1s_ref, b1s_ref, w2s_ref, b2s_ref, mu_ref, ls_ref):
    u = h_ref[...] + q0_ref[...] + q1_ref[...]
    tm = jnp.maximum(_dot(u, w1m_ref[...]) + b1m_ref[...], 0.0)
    mu_ref[...] = _dot(tm, w2m_ref[...]) + b2m_ref[...]
    ts = jnp.maximum(_dot(u, w1s_ref[...]) + b1s_ref[...], 0.0)
    ls_ref[...] = _dot(ts, w2s_ref[...]) + b2s_ref[...]
  n = h.shape[0]
  return pl.pallas_call(
      body,
      out_shape=(
          jax.ShapeDtypeStruct((n, w2m.shape[1]), jnp.float32),
          jax.ShapeDtypeStruct((n, w2s.shape[1]), jnp.float32),
      ),
  )(h, q0, q1, w1m, b1m, w2m, b2m, w1s, b1s, w2s, b2s)


@jax.jit
def kernel(x, edge_index, W1a, b1a, W2a, b2a, W1m, b1m, W2m, b2m,
           W1s, b1s, W2s, b2s):
  d_hid = W1a.shape[1]
  d_out = W2a.shape[1]
  srcs = edge_index[0].reshape(_NW, _K, _B)
  dsts = edge_index[1].reshape(_NW, _K, _B)

  z = _proj(x, W1a)
  p = _edge_agg(z, srcs, dsts, jnp.zeros((_RPS, d_hid), jnp.float32), d_hid)
  h = _mid(z, p[0], p[1], b1a.reshape(1, -1), W2a, b2a.reshape(1, -1))

  q = _edge_agg(h, srcs, dsts, jnp.zeros((_RPS, d_out), jnp.float32), d_out)
  mu, logstd = _heads(h, q[0], q[1],
                      W1m, b1m.reshape(1, -1), W2m, b2m.reshape(1, -1),
                      W1s, b1s.reshape(1, -1), W2s, b2s.reshape(1, -1))
  return (mu, logstd)


# speedup vs baseline: 16.3704x; 1.0520x over previous
import functools

import jax
import jax.numpy as jnp
from jax import lax
from jax.experimental import pallas as pl
from jax.experimental.pallas import tpu as pltpu
from jax.experimental.pallas import tpu_sc as plsc

_N = 10000
_E = 320000

_NC = 2
_NS = 16
_NW = _NC * _NS
_B = 125
_EW = _E // _NW
_K = _EW // _B
_RPS = _N // _NS


def _edge_agg(table, srcs, dsts, zeros, d):
  mesh = plsc.VectorSubcoreMesh(core_axis_name="c", subcore_axis_name="s")
  table = table.reshape(_NS, _RPS, d)

  @functools.partial(
      pl.kernel,
      out_type=jax.ShapeDtypeStruct((_NC, _NS, _RPS, d), jnp.float32),
      mesh=mesh,
      compiler_params=pltpu.CompilerParams(use_tc_tiling_on_sc=False),
      scratch_types=[
          pltpu.VMEM((_K, _B), jnp.int32),
          pltpu.VMEM((_K, _B), jnp.int32),
          pltpu.VMEM((4, _B, d), jnp.float32),
          pltpu.VMEM_SHARED((_N, d), jnp.float32),
          pltpu.VMEM_SHARED((_N, d), jnp.float32),
          pltpu.SemaphoreType.DMA((4,)),
          pltpu.SemaphoreType.DMA((4,)),
      ],
  )
  def agg(table_hbm, srcs_hbm, dsts_hbm, zeros_hbm, out_hbm,
          src_v, dst_v, rows_v, tbl_sh, acc_sh, gsem, ssem):
    c = lax.axis_index("c")
    s = lax.axis_index("s")
    wid = s * _NC + c

    pltpu.sync_copy(zeros_hbm, acc_sh.at[pl.ds(s * _RPS, _RPS)])
    pltpu.sync_copy(table_hbm.at[s], tbl_sh.at[pl.ds(s * _RPS, _RPS)])
    pltpu.sync_copy(srcs_hbm.at[wid], src_v)
    pltpu.sync_copy(dsts_hbm.at[wid], dst_v)
    plsc.subcore_barrier()

    def wait_dma(sem_slot, b):
      pltpu.make_async_copy(zeros_hbm.at[pl.ds(0, _B)], rows_v.at[b],
                            sem_slot).wait()

    for b in range(3):
      pltpu.async_copy(tbl_sh.at[src_v.at[b]], rows_v.at[b], gsem.at[b])

    @pl.loop(0, _K, step=4)
    def _(j):
      for b in range(4):
        jj = j + b
        wait_dma(gsem.at[b], b)
        pltpu.async_copy(rows_v.at[b], acc_sh.at[dst_v.at[jj]], ssem.at[b],
                         add=True)
        bb = (b + 3) % 4
        ja = jj + 3

        @pl.when(ja < _K)
        def _(bb=bb, ja=ja, jj=jj):
          @pl.when(jj >= 1)
          def _():
            wait_dma(ssem.at[bb], bb)
          pltpu.async_copy(tbl_sh.at[src_v.at[ja]], rows_v.at[bb],
                           gsem.at[bb])

    for b in range(4):
      wait_dma(ssem.at[b], b)
    plsc.subcore_barrier()
    pltpu.sync_copy(acc_sh.at[pl.ds(s * _RPS, _RPS)], out_hbm.at[c, s])

  return agg(table, srcs, dsts, zeros).reshape(_NC, _N, d)


def _dot(a, b):
  return jnp.dot(a, b, preferred_element_type=jnp.float32)


def _proj(x, w):
  def body(x_ref, w_ref, o_ref):
    o_ref[...] = _dot(x_ref[...], w_ref[...])
  return pl.pallas_call(
      body,
      out_shape=jax.ShapeDtypeStruct((x.shape[0], w.shape[1]), jnp.float32),
  )(x, w)


def _mid(z, p0, p1, b1, w2, b2):
  def body(z_ref, p0_ref, p1_ref, b1_ref, w2_ref, b2_ref, o_ref):
    t = jnp.maximum(z_ref[...] + p0_ref[...] + p1_ref[...] + b1_ref[...], 0.0)
    o_ref[...] = jnp.maximum(_dot(t, w2_ref[...]) + b2_ref[...], 0.0)
  return pl.pallas_call(
      body,
      out_shape=jax.ShapeDtypeStruct((z.shape[0], w2.shape[1]), jnp.float32),
  )(z, p0, p1, b1, w2, b2)


def _heads(h, q0, q1, w1m, b1m, w2m, b2m, w1s, b1s, w2s, b2s):
  def body(h_ref, q0_ref, q1_ref, w1m_ref, b1m_ref, w2m_ref, b2m_ref,
           w1s_ref, b1s_ref, w2s_ref, b2s_ref, mu_ref, ls_ref):
    u = h_ref[...] + q0_ref[...] + q1_ref[...]
    tm = jnp.maximum(_dot(u, w1m_ref[...]) + b1m_ref[...], 0.0)
    mu_ref[...] = _dot(tm, w2m_ref[...]) + b2m_ref[...]
    ts = jnp.maximum(_dot(u, w1s_ref[...]) + b1s_ref[...], 0.0)
    ls_ref[...] = _dot(ts, w2s_ref[...]) + b2s_ref[...]
  n = h.shape[0]
  return pl.pallas_call(
      body,
      out_shape=(
          jax.ShapeDtypeStruct((n, w2m.shape[1]), jnp.float32),
          jax.ShapeDtypeStruct((n, w2s.shape[1]), jnp.float32),
      ),
  )(h, q0, q1, w1m, b1m, w2m, b2m, w1s, b1s, w2s, b2s)


@jax.jit
def kernel(x, edge_index, W1a, b1a, W2a, b2a, W1m, b1m, W2m, b2m,
           W1s, b1s, W2s, b2s):
  d_hid = W1a.shape[1]
  d_out = W2a.shape[1]
  srcs = edge_index[0].reshape(_NW, _K, _B)
  dsts = edge_index[1].reshape(_NW, _K, _B)

  z = _proj(x, W1a)
  p = _edge_agg(z, srcs, dsts, jnp.zeros((_RPS, d_hid), jnp.float32), d_hid)
  h = _mid(z, p[0], p[1], b1a.reshape(1, -1), W2a, b2a.reshape(1, -1))

  q = _edge_agg(h, srcs, dsts, jnp.zeros((_RPS, d_out), jnp.float32), d_out)
  mu, logstd = _heads(h, q[0], q[1],
                      W1m, b1m.reshape(1, -1), W2m, b2m.reshape(1, -1),
                      W1s, b1s.reshape(1, -1), W2s, b2s.reshape(1, -1))
  return (mu, logstd)


# speedup vs baseline: 20.1010x; 1.2279x over previous
import functools

import jax
import jax.numpy as jnp
from jax import lax
from jax.experimental import pallas as pl
from jax.experimental.pallas import tpu as pltpu
from jax.experimental.pallas import tpu_sc as plsc

_N = 10000
_E = 320000

_NC = 2
_NS = 16
_NW = _NC * _NS
_B = 125
_EW = _E // _NW
_K = _EW // _B
_RPS = _N // _NS


def _edge_agg(table, srcs, dsts, zeros, d):
  mesh = plsc.VectorSubcoreMesh(core_axis_name="c", subcore_axis_name="s")
  table = table.reshape(_NS, _RPS, d)

  @functools.partial(
      pl.kernel,
      out_type=jax.ShapeDtypeStruct((_NC, _NS, _RPS, d), jnp.float32),
      mesh=mesh,
      compiler_params=pltpu.CompilerParams(use_tc_tiling_on_sc=False),
      scratch_types=[
          pltpu.VMEM((_K, _B), jnp.int32),
          pltpu.VMEM((_K, _B), jnp.int32),
          pltpu.VMEM((4, _B, d), jnp.float32),
          pltpu.VMEM_SHARED((_N, d), jnp.float32),
          pltpu.VMEM_SHARED((_N, d), jnp.float32),
          pltpu.SemaphoreType.DMA((4,)),
          pltpu.SemaphoreType.DMA((4,)),
      ],
  )
  def agg(table_hbm, srcs_hbm, dsts_hbm, zeros_hbm, out_hbm,
          src_v, dst_v, rows_v, tbl_sh, acc_sh, gsem, ssem):
    c = lax.axis_index("c")
    s = lax.axis_index("s")
    wid = s * _NC + c

    pltpu.sync_copy(zeros_hbm, acc_sh.at[pl.ds(s * _RPS, _RPS)])
    pltpu.sync_copy(table_hbm.at[s], tbl_sh.at[pl.ds(s * _RPS, _RPS)])
    pltpu.sync_copy(srcs_hbm.at[wid], src_v)
    pltpu.sync_copy(dsts_hbm.at[wid], dst_v)
    plsc.subcore_barrier()

    def wait_dma(sem_slot, b):
      pltpu.make_async_copy(zeros_hbm.at[pl.ds(0, _B)], rows_v.at[b],
                            sem_slot).wait()

    for b in range(3):
      pltpu.async_copy(tbl_sh.at[src_v.at[b]], rows_v.at[b], gsem.at[b])

    @pl.loop(0, _K, step=4)
    def _(j):
      for b in range(4):
        jj = j + b
        wait_dma(gsem.at[b], b)
        pltpu.async_copy(rows_v.at[b], acc_sh.at[dst_v.at[jj]], ssem.at[b],
                         add=True)
        bb = (b + 3) % 4
        ja = jj + 3

        @pl.when(ja < _K)
        def _(bb=bb, ja=ja, jj=jj):
          @pl.when(jj >= 1)
          def _():
            wait_dma(ssem.at[bb], bb)
          pltpu.async_copy(tbl_sh.at[src_v.at[ja]], rows_v.at[bb],
                           gsem.at[bb])

    for b in range(4):
      wait_dma(ssem.at[b], b)
    plsc.subcore_barrier()
    pltpu.sync_copy(acc_sh.at[pl.ds(s * _RPS, _RPS)], out_hbm.at[c, s])

  return agg(table, srcs, dsts, zeros).reshape(_NC, _N, d)


def _dot(a, b):
  return jnp.dot(a, b, preferred_element_type=jnp.float32)


def _proj(x, w1_blk4):
  def body(x_ref, w_ref, o_ref):
    x4 = jnp.reshape(x_ref[...], (_N // 4, 4 * 128))
    o_ref[...] = _dot(x4, w_ref[...])
  return pl.pallas_call(
      body,
      out_shape=jax.ShapeDtypeStruct((_N // 4, 128), jnp.float32),
  )(x, w1_blk4)


def _mid(z_p, p0, p1, b1_t4, w2_blk8, b2_t8):
  def body(z_ref, p0_ref, p1_ref, b1_ref, w2_ref, b2_ref, o_ref):
    t = jnp.maximum(z_ref[...] + p0_ref[...] + p1_ref[...] + b1_ref[...], 0.0)
    t8 = jnp.reshape(t, (_N // 8, 256))
    o_ref[...] = jnp.maximum(_dot(t8, w2_ref[...]) + b2_ref[...], 0.0)
  return pl.pallas_call(
      body,
      out_shape=jax.ShapeDtypeStruct((_N // 8, 128), jnp.float32),
  )(z_p, p0, p1, b1_t4, w2_blk8, b2_t8)


def _heads(h_p, q0, q1, w1m_blk8, b1m_t8, w2m_blk8, b2m_t8,
           w1s_blk8, b1s_t8, w2s_blk8, b2s_t8):
  def body(h_ref, q0_ref, q1_ref, w1m_ref, b1m_ref, w2m_ref, b2m_ref,
           w1s_ref, b1s_ref, w2s_ref, b2s_ref, mu_ref, ls_ref):
    u = h_ref[...] + q0_ref[...] + q1_ref[...]
    tm = jnp.maximum(_dot(u, w1m_ref[...]) + b1m_ref[...], 0.0)
    mu_ref[...] = _dot(tm, w2m_ref[...]) + b2m_ref[...]
    ts = jnp.maximum(_dot(u, w1s_ref[...]) + b1s_ref[...], 0.0)
    ls_ref[...] = _dot(ts, w2s_ref[...]) + b2s_ref[...]
  return pl.pallas_call(
      body,
      out_shape=(
          jax.ShapeDtypeStruct((_N // 8, 128), jnp.float32),
          jax.ShapeDtypeStruct((_N // 8, 128), jnp.float32),
      ),
  )(h_p, q0, q1, w1m_blk8, b1m_t8, w2m_blk8, b2m_t8,
    w1s_blk8, b1s_t8, w2s_blk8, b2s_t8)


def _blk(w, g):
  return jnp.kron(jnp.eye(g, dtype=jnp.float32), w)


@jax.jit
def kernel(x, edge_index, W1a, b1a, W2a, b2a, W1m, b1m, W2m, b2m,
           W1s, b1s, W2s, b2s):
  srcs = edge_index[0].reshape(_NW, _K, _B)
  dsts = edge_index[1].reshape(_NW, _K, _B)

  z_p = _proj(x, _blk(W1a, 4))
  p = _edge_agg(z_p.reshape(_NS, _RPS, 32), srcs, dsts,
                jnp.zeros((_RPS, 32), jnp.float32), 32)
  p_p = p.reshape(2, _N // 4, 128)
  h_p = _mid(z_p, p_p[0], p_p[1], jnp.tile(b1a, 4)[None],
             _blk(W2a, 8), jnp.tile(b2a, 8)[None])

  q = _edge_agg(h_p.reshape(_NS, _RPS, 16), srcs, dsts,
                jnp.zeros((_RPS, 16), jnp.float32), 16)
  q_p = q.reshape(2, _N // 8, 128)
  mu_p, ls_p = _heads(h_p, q_p[0], q_p[1],
                      _blk(W1m, 8), jnp.tile(b1m, 8)[None],
                      _blk(W2m, 8), jnp.tile(b2m, 8)[None],
                      _blk(W1s, 8), jnp.tile(b1s, 8)[None],
                      _blk(W2s, 8), jnp.tile(b2s, 8)[None])
  return (mu_p.reshape(_N, 16), ls_p.reshape(_N, 16))


# speedup vs baseline: 24.9870x; 1.2431x over previous
import functools

import jax
import jax.numpy as jnp
import numpy as np
from jax import lax
from jax.experimental import pallas as pl
from jax.experimental.pallas import tpu as pltpu
from jax.experimental.pallas import tpu_sc as plsc

_N = 10000
_E = 320000

_NC = 2
_NS = 16
_NW = _NC * _NS
_B = 125
_KA = _E // (_NS * _B)
_KB = _E // (_NW * _B)
_RPS = _N // _NS


def _edge_agg(table, srcs, dsts, zeros, feature_split):
  mesh = plsc.VectorSubcoreMesh(core_axis_name="c", subcore_axis_name="s")
  k = _KA if feature_split else _KB

  @functools.partial(
      pl.kernel,
      out_type=jax.ShapeDtypeStruct((_N, 32), jnp.float32),
      mesh=mesh,
      compiler_params=pltpu.CompilerParams(use_tc_tiling_on_sc=False),
      scratch_types=[
          pltpu.VMEM((k, _B), jnp.int32),
          pltpu.VMEM((k, _B), jnp.int32),
          pltpu.VMEM((4, _B, 16), jnp.float32),
          pltpu.VMEM_SHARED((_N, 16), jnp.float32),
          pltpu.VMEM_SHARED((_N, 16), jnp.float32),
          pltpu.SemaphoreType.DMA((4,)),
          pltpu.SemaphoreType.DMA((4,)),
      ],
  )
  def agg(table_hbm, srcs_hbm, dsts_hbm, zeros_hbm, out_hbm,
          src_v, dst_v, rows_v, tbl_sh, acc_sh, gsem, ssem):
    c = lax.axis_index("c")
    s = lax.axis_index("s")
    rows = pl.ds(s * _RPS, _RPS)

    pltpu.sync_copy(zeros_hbm, acc_sh.at[rows])
    if feature_split:
      pltpu.sync_copy(table_hbm.at[rows, pl.ds(16 * c, 16)], tbl_sh.at[rows])
      widx = s
    else:
      pltpu.sync_copy(table_hbm.at[rows], tbl_sh.at[rows])
      widx = s * _NC + c
    pltpu.sync_copy(srcs_hbm.at[widx], src_v)
    pltpu.sync_copy(dsts_hbm.at[widx], dst_v)
    plsc.subcore_barrier()

    def wait_dma(sem_slot, b):
      pltpu.make_async_copy(zeros_hbm.at[pl.ds(0, _B)], rows_v.at[b],
                            sem_slot).wait()

    for b in range(3):
      pltpu.async_copy(tbl_sh.at[src_v.at[b]], rows_v.at[b], gsem.at[b])

    @pl.loop(0, k, step=4)
    def _(j):
      for b in range(4):
        jj = j + b
        wait_dma(gsem.at[b], b)
        pltpu.async_copy(rows_v.at[b], acc_sh.at[dst_v.at[jj]], ssem.at[b],
                         add=True)
        bb = (b + 3) % 4
        ja = jj + 3

        @pl.when(ja < k)
        def _(bb=bb, ja=ja, jj=jj):
          @pl.when(jj >= 1)
          def _():
            wait_dma(ssem.at[bb], bb)
          pltpu.async_copy(tbl_sh.at[src_v.at[ja]], rows_v.at[bb],
                           gsem.at[bb])

    for b in range(4):
      wait_dma(ssem.at[b], b)
    plsc.subcore_barrier()
    pltpu.sync_copy(acc_sh.at[rows], out_hbm.at[rows, pl.ds(16 * c, 16)])

  return agg(table, srcs, dsts, zeros)


def _dot(a, b):
  return jnp.dot(a, b, preferred_element_type=jnp.float32)


def _proj(x, w1_blk4):
  def body(x_ref, w_ref, o_ref):
    x4 = jnp.reshape(x_ref[...], (_N // 4, 4 * 128))
    o_ref[...] = _dot(x4, w_ref[...])
  return pl.pallas_call(
      body,
      out_shape=jax.ShapeDtypeStruct((_N // 4, 128), jnp.float32),
  )(x, w1_blk4)


def _mid(z_p, p_p, b1_t4, w2_blk8, b2_t8):
  def body(z_ref, p_ref, b1_ref, w2_ref, b2_ref, o_ref):
    t = jnp.maximum(z_ref[...] + p_ref[...] + b1_ref[...], 0.0)
    t8 = jnp.reshape(t, (_N // 8, 256))
    o_ref[...] = jnp.maximum(_dot(t8, w2_ref[...]) + b2_ref[...], 0.0)
  return pl.pallas_call(
      body,
      out_shape=jax.ShapeDtypeStruct((_N // 8, 128), jnp.float32),
  )(z_p, p_p, b1_t4, w2_blk8, b2_t8)


def _heads(h_p, q_p, sel, w1m_blk8, b1m_t8, w2m_blk8, b2m_t8,
           w1s_blk8, b1s_t8, w2s_blk8, b2s_t8):
  def body(h_ref, q_ref, sel_ref, w1m_ref, b1m_ref, w2m_ref, b2m_ref,
           w1s_ref, b1s_ref, w2s_ref, b2s_ref, mu_ref, ls_ref):
    q8 = jnp.reshape(q_ref[...], (_N // 8, 256))
    u = h_ref[...] + _dot(q8, sel_ref[...])
    tm = jnp.maximum(_dot(u, w1m_ref[...]) + b1m_ref[...], 0.0)
    mu_ref[...] = _dot(tm, w2m_ref[...]) + b2m_ref[...]
    ts = jnp.maximum(_dot(u, w1s_ref[...]) + b1s_ref[...], 0.0)
    ls_ref[...] = _dot(ts, w2s_ref[...]) + b2s_ref[...]
  return pl.pallas_call(
      body,
      out_shape=(
          jax.ShapeDtypeStruct((_N // 8, 128), jnp.float32),
          jax.ShapeDtypeStruct((_N // 8, 128), jnp.float32),
      ),
  )(h_p, q_p, sel, w1m_blk8, b1m_t8, w2m_blk8, b2m_t8,
    w1s_blk8, b1s_t8, w2s_blk8, b2s_t8)


def _blk(w, g):
  return jnp.kron(jnp.eye(g, dtype=jnp.float32), w)


_SEL = np.zeros((256, 128), np.float32)
for _q in range(8):
  for _c in range(16):
    _SEL[32 * _q + _c, 16 * _q + _c] = 1.0
    _SEL[32 * _q + 16 + _c, 16 * _q + _c] = 1.0


@jax.jit
def kernel(x, edge_index, W1a, b1a, W2a, b2a, W1m, b1m, W2m, b2m,
           W1s, b1s, W2s, b2s):
  srcs_a = edge_index[0].reshape(_NS, _KA, _B)
  dsts_a = edge_index[1].reshape(_NS, _KA, _B)
  srcs_b = edge_index[0].reshape(_NW, _KB, _B)
  dsts_b = edge_index[1].reshape(_NW, _KB, _B)
  zeros = jnp.zeros((_RPS, 16), jnp.float32)

  z_p = _proj(x, _blk(W1a, 4))
  p = _edge_agg(z_p.reshape(_N, 32), srcs_a, dsts_a, zeros, True)
  h_p = _mid(z_p, p.reshape(_N // 4, 128), jnp.tile(b1a, 4)[None],
             _blk(W2a, 8), jnp.tile(b2a, 8)[None])

  q = _edge_agg(h_p.reshape(_N, 16), srcs_b, dsts_b, zeros, False)
  mu_p, ls_p = _heads(h_p, q.reshape(_N // 4, 128), jnp.asarray(_SEL),
                      _blk(W1m, 8), jnp.tile(b1m, 8)[None],
                      _blk(W2m, 8), jnp.tile(b2m, 8)[None],
                      _blk(W1s, 8), jnp.tile(b1s, 8)[None],
                      _blk(W2s, 8), jnp.tile(b2s, 8)[None])
  return (mu_p.reshape(_N, 16), ls_p.reshape(_N, 16))


# speedup vs baseline: 27.1463x; 1.0864x over previous
import functools

import jax
import jax.numpy as jnp
import numpy as np
from jax import lax
from jax.experimental import pallas as pl
from jax.experimental.pallas import tpu as pltpu
from jax.experimental.pallas import tpu_sc as plsc

_N = 10000
_E = 320000

_NC = 2
_NS = 16
_NW = _NC * _NS
_B = 125
_KA = _E // (_NS * _B)
_KB = _E // (_NW * _B)
_RPS = _N // _NS


def _edge_agg(table, ei3, zeros, feature_split):
  mesh = plsc.VectorSubcoreMesh(core_axis_name="c", subcore_axis_name="s")
  k = _KA if feature_split else _KB

  @functools.partial(
      pl.kernel,
      out_type=jax.ShapeDtypeStruct((_N, 32), jnp.float32),
      mesh=mesh,
      compiler_params=pltpu.CompilerParams(use_tc_tiling_on_sc=False),
      scratch_types=[
          pltpu.VMEM((k, _B), jnp.int32),
          pltpu.VMEM((k, _B), jnp.int32),
          pltpu.VMEM((4, _B, 16), jnp.float32),
          pltpu.VMEM_SHARED((_N, 16), jnp.float32),
          pltpu.VMEM_SHARED((_N, 16), jnp.float32),
          pltpu.SemaphoreType.DMA((4,)),
          pltpu.SemaphoreType.DMA((4,)),
      ],
  )
  def agg(table_hbm, ei_hbm, zeros_hbm, out_hbm,
          src_v, dst_v, rows_v, tbl_sh, acc_sh, gsem, ssem):
    c = lax.axis_index("c")
    s = lax.axis_index("s")
    rows = pl.ds(s * _RPS, _RPS)

    pltpu.sync_copy(zeros_hbm, acc_sh.at[rows])
    if feature_split:
      pltpu.sync_copy(table_hbm.at[rows, pl.ds(16 * c, 16)], tbl_sh.at[rows])
      widx = s
    else:
      pltpu.sync_copy(table_hbm.at[rows], tbl_sh.at[rows])
      widx = s * _NC + c
    pltpu.sync_copy(ei_hbm.at[0, pl.ds(widx * k, k)], src_v)
    pltpu.sync_copy(ei_hbm.at[1, pl.ds(widx * k, k)], dst_v)
    plsc.subcore_barrier()

    def wait_dma(sem_slot, b):
      pltpu.make_async_copy(zeros_hbm.at[pl.ds(0, _B)], rows_v.at[b],
                            sem_slot).wait()

    for b in range(3):
      pltpu.async_copy(tbl_sh.at[src_v.at[b]], rows_v.at[b], gsem.at[b])

    @pl.loop(0, k, step=4)
    def _(j):
      for b in range(4):
        jj = j + b
        wait_dma(gsem.at[b], b)
        pltpu.async_copy(rows_v.at[b], acc_sh.at[dst_v.at[jj]], ssem.at[b],
                         add=True)
        bb = (b + 3) % 4
        ja = jj + 3

        @pl.when(ja < k)
        def _(bb=bb, ja=ja, jj=jj):
          @pl.when(jj >= 1)
          def _():
            wait_dma(ssem.at[bb], bb)
          pltpu.async_copy(tbl_sh.at[src_v.at[ja]], rows_v.at[bb],
                           gsem.at[bb])

    for b in range(4):
      wait_dma(ssem.at[b], b)
    plsc.subcore_barrier()
    pltpu.sync_copy(acc_sh.at[rows], out_hbm.at[rows, pl.ds(16 * c, 16)])

  return agg(table, ei3, zeros)


def _dot(a, b):
  return jnp.dot(a, b, preferred_element_type=jnp.float32)


def _proj(x, w1_blk4):
  def body(x_ref, w_ref, o_ref):
    x4 = jnp.reshape(x_ref[...], (_N // 4, 4 * 128))
    o_ref[...] = _dot(x4, w_ref[...])
  return pl.pallas_call(
      body,
      out_shape=jax.ShapeDtypeStruct((_N // 4, 128), jnp.float32),
  )(x, w1_blk4)


def _mid(z_p, p_p, b1_t4, w2_blk8, b2_t8):
  def body(z_ref, p_ref, b1_ref, w2_ref, b2_ref, o_ref):
    t = jnp.maximum(z_ref[...] + p_ref[...] + b1_ref[...], 0.0)
    t8 = jnp.reshape(t, (_N // 8, 256))
    o_ref[...] = jnp.maximum(_dot(t8, w2_ref[...]) + b2_ref[...], 0.0)
  return pl.pallas_call(
      body,
      out_shape=jax.ShapeDtypeStruct((_N // 8, 128), jnp.float32),
  )(z_p, p_p, b1_t4, w2_blk8, b2_t8)


def _heads(h_p, q_p, sel, w1m_blk8, b1m_t8, w2m_blk8, b2m_t8,
           w1s_blk8, b1s_t8, w2s_blk8, b2s_t8):
  def body(h_ref, q_ref, sel_ref, w1m_ref, b1m_ref, w2m_ref, b2m_ref,
           w1s_ref, b1s_ref, w2s_ref, b2s_ref, mu_ref, ls_ref):
    q8 = jnp.reshape(q_ref[...], (_N // 8, 256))
    u = h_ref[...] + _dot(q8, sel_ref[...])
    tm = jnp.maximum(_dot(u, w1m_ref[...]) + b1m_ref[...], 0.0)
    mu_ref[...] = _dot(tm, w2m_ref[...]) + b2m_ref[...]
    ts = jnp.maximum(_dot(u, w1s_ref[...]) + b1s_ref[...], 0.0)
    ls_ref[...] = _dot(ts, w2s_ref[...]) + b2s_ref[...]
  return pl.pallas_call(
      body,
      out_shape=(
          jax.ShapeDtypeStruct((_N // 8, 128), jnp.float32),
          jax.ShapeDtypeStruct((_N // 8, 128), jnp.float32),
      ),
  )(h_p, q_p, sel, w1m_blk8, b1m_t8, w2m_blk8, b2m_t8,
    w1s_blk8, b1s_t8, w2s_blk8, b2s_t8)


def _blk(w, g):
  return jnp.kron(jnp.eye(g, dtype=jnp.float32), w)


_SEL = np.zeros((256, 128), np.float32)
for _q in range(8):
  for _c in range(16):
    _SEL[32 * _q + _c, 16 * _q + _c] = 1.0
    _SEL[32 * _q + 16 + _c, 16 * _q + _c] = 1.0


@jax.jit
def kernel(x, edge_index, W1a, b1a, W2a, b2a, W1m, b1m, W2m, b2m,
           W1s, b1s, W2s, b2s):
  ei3 = edge_index.reshape(2, _E // _B, _B)
  zeros = jnp.zeros((_RPS, 16), jnp.float32)

  z_p = _proj(x, _blk(W1a, 4))
  p = _edge_agg(z_p.reshape(_N, 32), ei3, zeros, True)
  h_p = _mid(z_p, p.reshape(_N // 4, 128), jnp.tile(b1a, 4)[None],
             _blk(W2a, 8), jnp.tile(b2a, 8)[None])

  q = _edge_agg(h_p.reshape(_N, 16), ei3, zeros, False)
  mu_p, ls_p = _heads(h_p, q.reshape(_N // 4, 128), jnp.asarray(_SEL),
                      _blk(W1m, 8), jnp.tile(b1m, 8)[None],
                      _blk(W2m, 8), jnp.tile(b2m, 8)[None],
                      _blk(W1s, 8), jnp.tile(b1s, 8)[None],
                      _blk(W2s, 8), jnp.tile(b2s, 8)[None])
  return (mu_p.reshape(_N, 16), ls_p.reshape(_N, 16))
